# Initial kernel scaffold; baseline (speedup 1.0000x reference)
import jax
import jax.numpy as jnp
from jax.experimental import pallas as pl


def kernel(x, edge_index, batch, W1, b1, W2, b2, Wlin, blin):
    raise NotImplementedError("write your pallas kernel here")


# speedup vs baseline: 63.4592x; 63.4592x over previous
import functools

import jax
import jax.numpy as jnp
import numpy as np
from jax import lax
from jax.experimental import pallas as pl
from jax.experimental.pallas import tpu as pltpu
from jax.experimental.pallas import tpu_sc as plsc

N = 100000
E = 3200000
F = 16
G = 64
CLS = 7
NC = 2
NS = 16
NW = NC * NS
CH = 128
EWR = E // NW
CUT = 776 * CH
NCHUNK = 800
EW = NCHUNK * CH
E_PAD = EW * NW
PAD_A = 6 * CH
REAL_B = EWR - CUT
PAD_TAIL = EW - CUT - PAD_A - REAL_B
ACC_ROWS = 100096
ZR = ACC_ROWS // NS
PAD_ROWS = ACC_ROWS - N

KB = 8
NOB = NCHUNK // KB
NROWS2 = E_PAD // CH

NP = N // 8
ACCP = ACC_ROWS // 8
RP = 256
NBLK = -(-NP // RP)
NPB = NBLK * RP * 8


def _sc_mesh():
    return plsc.VectorSubcoreMesh(core_axis_name="c", subcore_axis_name="s")


def _sc_degree(dstp, zeros_acc, ones_rows):
    @functools.partial(
        pl.kernel,
        out_type=jax.ShapeDtypeStruct((NC * ACC_ROWS, F), jnp.float32),
        mesh=_sc_mesh(),
        compiler_params=pltpu.CompilerParams(use_tc_tiling_on_sc=False),
        scratch_types=[
            pltpu.VMEM((2, KB, CH), jnp.int32),
            pltpu.VMEM((CH, F), jnp.float32),
            pltpu.VMEM((CH, F), jnp.float32),
            pltpu.VMEM_SHARED((ACC_ROWS, F), jnp.float32),
            pltpu.SemaphoreType.DMA,
            pltpu.SemaphoreType.DMA((KB,)),
        ],
    )
    def kern(dst_hbm, zero_hbm, ones_hbm, out_hbm, didx2, ones_v, zeros_v,
             acc_sh, isem, ssem):
        cid = lax.axis_index("c")
        sid = lax.axis_index("s")
        w = sid * NC + cid
        pltpu.sync_copy(zero_hbm.at[pl.ds(sid * ZR, ZR)],
                        acc_sh.at[pl.ds(sid * ZR, ZR)])
        pltpu.sync_copy(ones_hbm, ones_v)
        pltpu.sync_copy(zero_hbm.at[pl.ds(0, CH)], zeros_v)
        plsc.subcore_barrier()
        crow = w * NCHUNK
        pltpu.sync_copy(dst_hbm.at[pl.ds(crow, KB)], didx2.at[0])

        @pl.loop(0, NOB)
        def _(j):
            p = lax.rem(j, 2)
            nrow = crow + lax.rem(j + 1, NOB) * KB
            pf = pltpu.async_copy(dst_hbm.at[pl.ds(nrow, KB)],
                                  didx2.at[1 - p], isem)
            hs = []
            for b in range(KB):
                hs.append(pltpu.async_copy(
                    ones_v, acc_sh.at[didx2.at[p, b]], ssem.at[b], add=True))
            for b in range(KB):
                hs[b].wait()
            pf.wait()

        for t in range(2 * KB):
            pltpu.sync_copy(zeros_v, acc_sh.at[didx2.at[0, t % KB]],
                            add=True)

        plsc.subcore_barrier()
        pltpu.sync_copy(acc_sh.at[pl.ds(sid * ZR, ZR)],
                        out_hbm.at[pl.ds(cid * ACC_ROWS + sid * ZR, ZR)])

    return kern(dstp, zeros_acc, ones_rows)


def _sc_scatter(y, srcp, dstp, zeros_acc):
    @functools.partial(
        pl.kernel,
        out_type=jax.ShapeDtypeStruct((NC * ACC_ROWS, F), jnp.float32),
        mesh=_sc_mesh(),
        compiler_params=pltpu.CompilerParams(use_tc_tiling_on_sc=False),
        scratch_types=[
            pltpu.VMEM((2, KB, CH), jnp.int32),
            pltpu.VMEM((2, KB, CH), jnp.int32),
            pltpu.VMEM((KB, CH, F), jnp.float32),
            pltpu.VMEM((CH, F), jnp.float32),
            pltpu.VMEM_SHARED((ACC_ROWS, F), jnp.float32),
            pltpu.SemaphoreType.DMA((2,)),
            pltpu.SemaphoreType.DMA((KB,)),
            pltpu.SemaphoreType.DMA((KB,)),
        ],
    )
    def kern(y_hbm, src_hbm, dst_hbm, zero_hbm, out_hbm,
             sidx2, didx2, rows2, zeros_v, acc_sh, isem, gsem, ssem):
        cid = lax.axis_index("c")
        sid = lax.axis_index("s")
        w = sid * NC + cid
        pltpu.sync_copy(zero_hbm.at[pl.ds(sid * ZR, ZR)],
                        acc_sh.at[pl.ds(sid * ZR, ZR)])
        pltpu.sync_copy(zero_hbm.at[pl.ds(0, CH)], zeros_v)
        plsc.subcore_barrier()
        crow = w * NCHUNK
        pltpu.sync_copy(src_hbm.at[pl.ds(crow, KB)], sidx2.at[0])
        pltpu.sync_copy(dst_hbm.at[pl.ds(crow, KB)], didx2.at[0])

        @pl.loop(0, NOB)
        def _(j):
            p = lax.rem(j, 2)
            nrow = crow + lax.rem(j + 1, NOB) * KB
            pf_s = pltpu.async_copy(src_hbm.at[pl.ds(nrow, KB)],
                                    sidx2.at[1 - p], isem.at[0])
            pf_d = pltpu.async_copy(dst_hbm.at[pl.ds(nrow, KB)],
                                    didx2.at[1 - p], isem.at[1])
            ghs = []
            for b in range(KB):
                ghs.append(pltpu.async_copy(
                    y_hbm.at[sidx2.at[p, b]], rows2.at[b], gsem.at[b]))
            shs = []
            for b in range(KB):
                ghs[b].wait()
                shs.append(pltpu.async_copy(
                    rows2.at[b], acc_sh.at[didx2.at[p, b]], ssem.at[b],
                    add=True))
            for b in range(KB):
                shs[b].wait()
            pf_s.wait()
            pf_d.wait()

        for t in range(2 * KB):
            pltpu.sync_copy(zeros_v, acc_sh.at[didx2.at[0, t % KB]],
                            add=True)

        plsc.subcore_barrier()
        pltpu.sync_copy(acc_sh.at[pl.ds(sid * ZR, ZR)],
                        out_hbm.at[pl.ds(cid * ACC_ROWS + sid * ZR, ZR)])

    return kern(y, srcp, dstp, zeros_acc)


def _tc_xw1(xp, W1b):
    def body(x_ref, w_ref, o_ref):
        o_ref[...] = jnp.dot(x_ref[...], w_ref[...],
                             preferred_element_type=jnp.float32,
                             precision=lax.Precision.HIGHEST)

    return pl.pallas_call(
        body,
        grid=(NBLK,),
        in_specs=[
            pl.BlockSpec((RP, 128), lambda i: (i, 0)),
            pl.BlockSpec((128, 128), lambda i: (0, 0)),
        ],
        out_specs=pl.BlockSpec((RP, 128), lambda i: (i, 0)),
        out_shape=jax.ShapeDtypeStruct((NP, 128), jnp.float32),
    )(xp, W1b)


def _tc_prep(d0p, d1p, xw1p):
    def body(d0_ref, d1_ref, xw_ref, dinv_ref, y1_ref):
        deg = d0_ref[...] + d1_ref[...] + 1.0
        dinv = lax.rsqrt(deg)
        dinv_ref[...] = dinv
        y1_ref[...] = dinv * xw_ref[...]

    return pl.pallas_call(
        body,
        grid=(NBLK,),
        in_specs=[
            pl.BlockSpec((RP, 128), lambda i: (i, 0)),
            pl.BlockSpec((RP, 128), lambda i: (i, 0)),
            pl.BlockSpec((RP, 128), lambda i: (i, 0)),
        ],
        out_specs=[
            pl.BlockSpec((RP, 128), lambda i: (i, 0)),
            pl.BlockSpec((RP, 128), lambda i: (i, 0)),
        ],
        out_shape=[
            jax.ShapeDtypeStruct((NP, 128), jnp.float32),
            jax.ShapeDtypeStruct((NP, 128), jnp.float32),
        ],
    )(d0p, d1p, xw1p)


def _tc_mid(a0p, a1p, y1p, dinvp, W2b, b1t):
    def body(a0_ref, a1_ref, y1_ref, dinv_ref, w_ref, b_ref, y2_ref):
        dinv = dinv_ref[...]
        h1 = jnp.maximum(
            dinv * (a0_ref[...] + a1_ref[...] + y1_ref[...]) + b_ref[...],
            0.0)
        y2_ref[...] = dinv * jnp.dot(h1, w_ref[...],
                                     preferred_element_type=jnp.float32,
                                     precision=lax.Precision.HIGHEST)

    return pl.pallas_call(
        body,
        grid=(NBLK,),
        in_specs=[
            pl.BlockSpec((RP, 128), lambda i: (i, 0)),
            pl.BlockSpec((RP, 128), lambda i: (i, 0)),
            pl.BlockSpec((RP, 128), lambda i: (i, 0)),
            pl.BlockSpec((RP, 128), lambda i: (i, 0)),
            pl.BlockSpec((128, 128), lambda i: (0, 0)),
            pl.BlockSpec((1, 128), lambda i: (0, 0)),
        ],
        out_specs=pl.BlockSpec((RP, 128), lambda i: (i, 0)),
        out_shape=jax.ShapeDtypeStruct((NP, 128), jnp.float32),
    )(a0p, a1p, y1p, dinvp, W2b, b1t)


def _tc_final(a0p, a1p, y2p, dinvp, b2t, batR, Wlin, blin):
    def body(a0_ref, a1_ref, y2_ref, dinv_ref, b_ref, bat_ref, wlin_ref,
             blin_ref, out_ref, pool_ref):
        i = pl.program_id(0)

        @pl.when(i == 0)
        def _():
            pool_ref[...] = jnp.zeros((G, F), jnp.float32)

        h2 = dinv_ref[...] * (a0_ref[...] + a1_ref[...] + y2_ref[...]) \
            + b_ref[...]
        rid = lax.broadcasted_iota(jnp.int32, (RP, 1), 0) + i * RP
        h2 = jnp.where(rid < NP, h2, 0.0)
        bat = bat_ref[...]
        giota = lax.broadcasted_iota(jnp.int32, (G, 1), 0)
        acc = jnp.zeros((G, F), jnp.float32)
        for a in range(8):
            oh = (bat[0, a:a + 1, :] == giota).astype(jnp.float32)
            acc = acc + jnp.dot(oh, h2[:, 16 * a:16 * a + 16],
                                preferred_element_type=jnp.float32,
                                precision=lax.Precision.HIGHEST)
        pool_ref[...] += acc

        @pl.when(i == NBLK - 1)
        def _():
            logits = jnp.dot(pool_ref[...], wlin_ref[...],
                             preferred_element_type=jnp.float32,
                             precision=lax.Precision.HIGHEST) + blin_ref[...]
            m = jnp.max(logits, axis=1, keepdims=True)
            s = jnp.sum(jnp.exp(logits - m), axis=1, keepdims=True)
            out_ref[...] = logits - m - jnp.log(s)

    return pl.pallas_call(
        body,
        grid=(NBLK,),
        in_specs=[
            pl.BlockSpec((RP, 128), lambda i: (i, 0)),
            pl.BlockSpec((RP, 128), lambda i: (i, 0)),
            pl.BlockSpec((RP, 128), lambda i: (i, 0)),
            pl.BlockSpec((RP, 128), lambda i: (i, 0)),
            pl.BlockSpec((1, 128), lambda i: (0, 0)),
            pl.BlockSpec((1, 8, RP), lambda i: (i, 0, 0)),
            pl.BlockSpec((F, CLS), lambda i: (0, 0)),
            pl.BlockSpec((1, CLS), lambda i: (0, 0)),
        ],
        out_specs=pl.BlockSpec((G, CLS), lambda i: (0, 0)),
        out_shape=jax.ShapeDtypeStruct((G, CLS), jnp.float32),
        scratch_shapes=[pltpu.VMEM((G, F), jnp.float32)],
    )(a0p, a1p, y2p, dinvp, b2t, batR, Wlin, blin)


def kernel(x, edge_index, batch, W1, b1, W2, b2, Wlin, blin):
    def lay_out(idx, pad_vals):
        iw = idx.reshape(NW, EWR)
        pa = jnp.broadcast_to(pad_vals[:PAD_A], (NW, PAD_A))
        pt = jnp.broadcast_to(pad_vals[PAD_A:PAD_A + PAD_TAIL],
                              (NW, PAD_TAIL))
        return jnp.concatenate(
            [iw[:, :CUT], pa, iw[:, CUT:], pt], axis=1).reshape(NROWS2, CH)

    src = edge_index[0]
    dst = edge_index[1]
    padi = jnp.arange(PAD_A + PAD_TAIL, dtype=jnp.int32)
    srcp = lay_out(src, padi % np.int32(N))
    dstp = lay_out(dst, np.int32(N) + padi % np.int32(PAD_ROWS))
    zeros_acc = jnp.zeros((ACC_ROWS, F), jnp.float32)
    ones_rows = jnp.ones((CH, F), jnp.float32)

    eye8 = jnp.eye(8, dtype=jnp.float32)
    xp = jnp.pad(x, ((0, 0), (0, F - 3))).reshape(NP, 128)
    W1b = jnp.kron(eye8, jnp.pad(W1, ((0, F - 3), (0, 0))))
    W2b = jnp.kron(eye8, W2)
    b1t = jnp.tile(b1, 8).reshape(1, 128)
    b2t = jnp.tile(b2, 8).reshape(1, 128)
    batR = jnp.pad(batch, (0, NPB - N), constant_values=G) \
        .reshape(NBLK, RP, 8).transpose(0, 2, 1)

    degp = _sc_degree(dstp, zeros_acc, ones_rows).reshape(NC * ACCP, 128)
    xw1p = _tc_xw1(xp, W1b)
    dinvp, y1p = _tc_prep(degp[:NP], degp[ACCP:ACCP + NP], xw1p)

    acc1p = _sc_scatter(y1p.reshape(N, F), srcp, dstp,
                        zeros_acc).reshape(NC * ACCP, 128)
    y2p = _tc_mid(acc1p[:NP], acc1p[ACCP:ACCP + NP], y1p, dinvp, W2b, b1t)

    acc2p = _sc_scatter(y2p.reshape(N, F), srcp, dstp,
                        zeros_acc).reshape(NC * ACCP, 128)
    return _tc_final(acc2p[:NP], acc2p[ACCP:ACCP + NP], y2p, dinvp, b2t, batR,
                     Wlin, blin.reshape(1, CLS))


# speedup vs baseline: 92.9867x; 1.4653x over previous
import functools

import jax
import jax.numpy as jnp
import numpy as np
from jax import lax
from jax.experimental import pallas as pl
from jax.experimental.pallas import tpu as pltpu
from jax.experimental.pallas import tpu_sc as plsc

N = 100000
E = 3200000
F = 16
G = 64
CLS = 7
NC = 2
NS = 16
NW = NC * NS
CH = 128
EWR = E // NW
CUT = 776 * CH
NCHUNK = 800
EW = NCHUNK * CH
E_PAD = EW * NW
PAD_A = 6 * CH
REAL_B = EWR - CUT
PAD_TAIL = EW - CUT - PAD_A - REAL_B
ACC_ROWS = 100096
ZR = ACC_ROWS // NS
PAD_ROWS = ACC_ROWS - N

KB = 8
NOB = NCHUNK // KB
NROWS2 = E_PAD // CH

NP = N // 8
ACCP = ACC_ROWS // 8
RP = 368
NBLK = ACCP // RP
NPB = NBLK * RP * 8


def _sc_mesh():
    return plsc.VectorSubcoreMesh(core_axis_name="c", subcore_axis_name="s")


def _sc_degree(dstp, zeros_acc, ones_rows):
    @functools.partial(
        pl.kernel,
        out_type=jax.ShapeDtypeStruct((NC * ACC_ROWS, F), jnp.float32),
        mesh=_sc_mesh(),
        compiler_params=pltpu.CompilerParams(use_tc_tiling_on_sc=False),
        scratch_types=[
            pltpu.VMEM((2, KB, CH), jnp.int32),
            pltpu.VMEM((CH, F), jnp.float32),
            pltpu.VMEM((CH, F), jnp.float32),
            pltpu.VMEM_SHARED((ACC_ROWS, F), jnp.float32),
            pltpu.SemaphoreType.DMA,
            pltpu.SemaphoreType.DMA((KB,)),
        ],
    )
    def kern(dst_hbm, zero_hbm, ones_hbm, out_hbm, didx2, ones_v, zeros_v,
             acc_sh, isem, ssem):
        cid = lax.axis_index("c")
        sid = lax.axis_index("s")
        w = sid * NC + cid
        pltpu.sync_copy(zero_hbm.at[pl.ds(sid * ZR, ZR)],
                        acc_sh.at[pl.ds(sid * ZR, ZR)])
        pltpu.sync_copy(ones_hbm, ones_v)
        pltpu.sync_copy(zero_hbm.at[pl.ds(0, CH)], zeros_v)
        plsc.subcore_barrier()
        crow = w * NCHUNK
        pltpu.sync_copy(dst_hbm.at[pl.ds(crow, KB)], didx2.at[0])

        @pl.loop(0, NOB)
        def _(j):
            p = lax.rem(j, 2)
            nrow = crow + lax.rem(j + 1, NOB) * KB
            pf = pltpu.async_copy(dst_hbm.at[pl.ds(nrow, KB)],
                                  didx2.at[1 - p], isem)
            hs = []
            for b in range(KB):
                hs.append(pltpu.async_copy(
                    ones_v, acc_sh.at[didx2.at[p, b]], ssem.at[b], add=True))
            for b in range(KB):
                hs[b].wait()
            pf.wait()

        for t in range(2 * KB):
            pltpu.sync_copy(zeros_v, acc_sh.at[didx2.at[0, t % KB]],
                            add=True)

        plsc.subcore_barrier()
        pltpu.sync_copy(acc_sh.at[pl.ds(sid * ZR, ZR)],
                        out_hbm.at[pl.ds(cid * ACC_ROWS + sid * ZR, ZR)])

    return kern(dstp, zeros_acc, ones_rows)


def _sc_scatter(y, srcp, dstp, zeros_acc):
    @functools.partial(
        pl.kernel,
        out_type=jax.ShapeDtypeStruct((NC * ACC_ROWS, F), jnp.float32),
        mesh=_sc_mesh(),
        compiler_params=pltpu.CompilerParams(use_tc_tiling_on_sc=False),
        scratch_types=[
            pltpu.VMEM((2, KB, CH), jnp.int32),
            pltpu.VMEM((2, KB, CH), jnp.int32),
            pltpu.VMEM((KB, CH, F), jnp.float32),
            pltpu.VMEM((CH, F), jnp.float32),
            pltpu.VMEM_SHARED((ACC_ROWS, F), jnp.float32),
            pltpu.SemaphoreType.DMA((2,)),
            pltpu.SemaphoreType.DMA((KB,)),
            pltpu.SemaphoreType.DMA((KB,)),
        ],
    )
    def kern(y_hbm, src_hbm, dst_hbm, zero_hbm, out_hbm,
             sidx2, didx2, rows2, zeros_v, acc_sh, isem, gsem, ssem):
        cid = lax.axis_index("c")
        sid = lax.axis_index("s")
        w = sid * NC + cid
        pltpu.sync_copy(zero_hbm.at[pl.ds(sid * ZR, ZR)],
                        acc_sh.at[pl.ds(sid * ZR, ZR)])
        pltpu.sync_copy(zero_hbm.at[pl.ds(0, CH)], zeros_v)
        plsc.subcore_barrier()
        crow = w * NCHUNK
        pltpu.sync_copy(src_hbm.at[pl.ds(crow, KB)], sidx2.at[0])
        pltpu.sync_copy(dst_hbm.at[pl.ds(crow, KB)], didx2.at[0])

        @pl.loop(0, NOB)
        def _(j):
            p = lax.rem(j, 2)
            nrow = crow + lax.rem(j + 1, NOB) * KB
            pf_s = pltpu.async_copy(src_hbm.at[pl.ds(nrow, KB)],
                                    sidx2.at[1 - p], isem.at[0])
            pf_d = pltpu.async_copy(dst_hbm.at[pl.ds(nrow, KB)],
                                    didx2.at[1 - p], isem.at[1])
            ghs = []
            for b in range(KB):
                ghs.append(pltpu.async_copy(
                    y_hbm.at[sidx2.at[p, b]], rows2.at[b], gsem.at[b]))
            shs = []
            for b in range(KB):
                ghs[b].wait()
                shs.append(pltpu.async_copy(
                    rows2.at[b], acc_sh.at[didx2.at[p, b]], ssem.at[b],
                    add=True))
            for b in range(KB):
                shs[b].wait()
            pf_s.wait()
            pf_d.wait()

        for t in range(2 * KB):
            pltpu.sync_copy(zeros_v, acc_sh.at[didx2.at[0, t % KB]],
                            add=True)

        plsc.subcore_barrier()
        pltpu.sync_copy(acc_sh.at[pl.ds(sid * ZR, ZR)],
                        out_hbm.at[pl.ds(cid * ACC_ROWS + sid * ZR, ZR)])

    return kern(y, srcp, dstp, zeros_acc)


def _tc_xw1(xp, W1b):
    def body(x_ref, w_ref, o_ref):
        o_ref[...] = jnp.dot(x_ref[...], w_ref[...],
                             preferred_element_type=jnp.float32,
                             precision=lax.Precision.HIGHEST)

    return pl.pallas_call(
        body,
        grid=(NBLK,),
        in_specs=[
            pl.BlockSpec((RP, 128), lambda i: (i, 0)),
            pl.BlockSpec((128, 128), lambda i: (0, 0)),
        ],
        out_specs=pl.BlockSpec((RP, 128), lambda i: (i, 0)),
        out_shape=jax.ShapeDtypeStruct((NP, 128), jnp.float32),
    )(xp, W1b)


def _tc_prep(d0p, d1p, xw1p):
    def body(d0_ref, d1_ref, xw_ref, dinv_ref, y1_ref):
        deg = d0_ref[...] + d1_ref[...] + 1.0
        dinv = lax.rsqrt(deg)
        dinv_ref[...] = dinv
        y1_ref[...] = dinv * xw_ref[...]

    return pl.pallas_call(
        body,
        grid=(NBLK,),
        in_specs=[
            pl.BlockSpec((RP, 128), lambda i: (i, 0)),
            pl.BlockSpec((RP, 128), lambda i: (i + NBLK, 0)),
            pl.BlockSpec((RP, 128), lambda i: (i, 0)),
        ],
        out_specs=[
            pl.BlockSpec((RP, 128), lambda i: (i, 0)),
            pl.BlockSpec((RP, 128), lambda i: (i, 0)),
        ],
        out_shape=[
            jax.ShapeDtypeStruct((NP, 128), jnp.float32),
            jax.ShapeDtypeStruct((NP, 128), jnp.float32),
        ],
    )(d0p, d1p, xw1p)


def _tc_mid(a0p, a1p, y1p, dinvp, W2b, b1t):
    def body(a0_ref, a1_ref, y1_ref, dinv_ref, w_ref, b_ref, y2_ref):
        dinv = dinv_ref[...]
        h1 = jnp.maximum(
            dinv * (a0_ref[...] + a1_ref[...] + y1_ref[...]) + b_ref[...],
            0.0)
        y2_ref[...] = dinv * jnp.dot(h1, w_ref[...],
                                     preferred_element_type=jnp.float32,
                                     precision=lax.Precision.HIGHEST)

    return pl.pallas_call(
        body,
        grid=(NBLK,),
        in_specs=[
            pl.BlockSpec((RP, 128), lambda i: (i, 0)),
            pl.BlockSpec((RP, 128), lambda i: (i + NBLK, 0)),
            pl.BlockSpec((RP, 128), lambda i: (i, 0)),
            pl.BlockSpec((RP, 128), lambda i: (i, 0)),
            pl.BlockSpec((128, 128), lambda i: (0, 0)),
            pl.BlockSpec((1, 128), lambda i: (0, 0)),
        ],
        out_specs=pl.BlockSpec((RP, 128), lambda i: (i, 0)),
        out_shape=jax.ShapeDtypeStruct((NP, 128), jnp.float32),
    )(a0p, a1p, y1p, dinvp, W2b, b1t)


def _tc_final(a0p, a1p, y2p, dinvp, b2t, batR, Wlin, blin):
    def body(a0_ref, a1_ref, y2_ref, dinv_ref, b_ref, bat_ref, wlin_ref,
             blin_ref, out_ref, pool_ref):
        i = pl.program_id(0)

        @pl.when(i == 0)
        def _():
            pool_ref[...] = jnp.zeros((G, F), jnp.float32)

        h2 = dinv_ref[...] * (a0_ref[...] + a1_ref[...] + y2_ref[...]) \
            + b_ref[...]
        rid = lax.broadcasted_iota(jnp.int32, (RP, 1), 0) + i * RP
        h2 = jnp.where(rid < NP, h2, 0.0)
        bat = bat_ref[...]
        giota = lax.broadcasted_iota(jnp.int32, (G, 1), 0)
        acc = jnp.zeros((G, F), jnp.float32)
        for a in range(8):
            oh = (bat[0, a:a + 1, :] == giota).astype(jnp.float32)
            acc = acc + jnp.dot(oh, h2[:, 16 * a:16 * a + 16],
                                preferred_element_type=jnp.float32,
                                precision=lax.Precision.HIGHEST)
        pool_ref[...] += acc

        @pl.when(i == NBLK - 1)
        def _():
            logits = jnp.dot(pool_ref[...], wlin_ref[...],
                             preferred_element_type=jnp.float32,
                             precision=lax.Precision.HIGHEST) + blin_ref[...]
            m = jnp.max(logits, axis=1, keepdims=True)
            s = jnp.sum(jnp.exp(logits - m), axis=1, keepdims=True)
            out_ref[...] = logits - m - jnp.log(s)

    return pl.pallas_call(
        body,
        grid=(NBLK,),
        in_specs=[
            pl.BlockSpec((RP, 128), lambda i: (i, 0)),
            pl.BlockSpec((RP, 128), lambda i: (i + NBLK, 0)),
            pl.BlockSpec((RP, 128), lambda i: (i, 0)),
            pl.BlockSpec((RP, 128), lambda i: (i, 0)),
            pl.BlockSpec((1, 128), lambda i: (0, 0)),
            pl.BlockSpec((1, 8, RP), lambda i: (i, 0, 0)),
            pl.BlockSpec((F, CLS), lambda i: (0, 0)),
            pl.BlockSpec((1, CLS), lambda i: (0, 0)),
        ],
        out_specs=pl.BlockSpec((G, CLS), lambda i: (0, 0)),
        out_shape=jax.ShapeDtypeStruct((G, CLS), jnp.float32),
        scratch_shapes=[pltpu.VMEM((G, F), jnp.float32)],
    )(a0p, a1p, y2p, dinvp, b2t, batR, Wlin, blin)


def kernel(x, edge_index, batch, W1, b1, W2, b2, Wlin, blin):
    def lay_out(idx, pad_vals):
        iw = idx.reshape(NW, EWR)
        pa = jnp.broadcast_to(pad_vals[:PAD_A], (NW, PAD_A))
        pt = jnp.broadcast_to(pad_vals[PAD_A:PAD_A + PAD_TAIL],
                              (NW, PAD_TAIL))
        return jnp.concatenate(
            [iw[:, :CUT], pa, iw[:, CUT:], pt], axis=1).reshape(NROWS2, CH)

    src = edge_index[0]
    dst = edge_index[1]
    padi = jnp.arange(PAD_A + PAD_TAIL, dtype=jnp.int32)
    srcp = lay_out(src, padi % np.int32(N))
    dstp = lay_out(dst, np.int32(N) + padi % np.int32(PAD_ROWS))
    zeros_acc = jnp.zeros((ACC_ROWS, F), jnp.float32)
    ones_rows = jnp.ones((CH, F), jnp.float32)

    eye8 = jnp.eye(8, dtype=jnp.float32)
    xp = jnp.pad(x, ((0, 0), (0, F - 3))).reshape(NP, 128)
    W1b = jnp.kron(eye8, jnp.pad(W1, ((0, F - 3), (0, 0))))
    W2b = jnp.kron(eye8, W2)
    b1t = jnp.tile(b1, 8).reshape(1, 128)
    b2t = jnp.tile(b2, 8).reshape(1, 128)
    batR = jnp.pad(batch, (0, NPB - N), constant_values=G) \
        .reshape(NBLK, RP, 8).transpose(0, 2, 1)

    degp = _sc_degree(dstp, zeros_acc, ones_rows).reshape(NC * ACCP, 128)
    xw1p = _tc_xw1(xp, W1b)
    dinvp, y1p = _tc_prep(degp, degp, xw1p)

    acc1p = _sc_scatter(y1p.reshape(N, F), srcp, dstp,
                        zeros_acc).reshape(NC * ACCP, 128)
    y2p = _tc_mid(acc1p, acc1p, y1p, dinvp, W2b, b1t)

    acc2p = _sc_scatter(y2p.reshape(N, F), srcp, dstp,
                        zeros_acc).reshape(NC * ACCP, 128)
    return _tc_final(acc2p, acc2p, y2p, dinvp, b2t, batR,
                     Wlin, blin.reshape(1, CLS))
